# Initial kernel scaffold; baseline (speedup 1.0000x reference)
import jax
import jax.numpy as jnp
from jax.experimental import pallas as pl


def kernel(x, edge_index, params):
    raise NotImplementedError("write your pallas kernel here")


# speedup vs baseline: 1.0784x; 1.0784x over previous
import functools
import jax
import jax.numpy as jnp
from jax.experimental import pallas as pl
from jax.experimental.pallas import tpu as pltpu

N = 10000
E = 160000
CH = 128
H = 8


def _matmul_kernel(a_ref, b_ref, o_ref):
    o_ref[...] = jnp.dot(a_ref[...], b_ref[...],
                         preferred_element_type=jnp.float32)


def _mm(a, b, block_m=1000):
    m, k = a.shape
    k2, n = b.shape
    grid = (m // block_m,)
    return pl.pallas_call(
        _matmul_kernel,
        grid=grid,
        in_specs=[
            pl.BlockSpec((block_m, k), lambda i: (i, 0)),
            pl.BlockSpec((k, n), lambda i: (0, 0)),
        ],
        out_specs=pl.BlockSpec((block_m, n), lambda i: (i, 0)),
        out_shape=jax.ShapeDtypeStruct((m, n), jnp.float32),
    )(a, b)


def _bn_relu(h, g, b):
    m = h.mean(axis=0)
    v = h.var(axis=0)
    return jnp.maximum((h - m) / jnp.sqrt(v + 1e-5) * g + b, 0.0)


def kernel(x, edge_index, params):
    src = edge_index[0]
    dst = edge_index[1]
    p = params

    feats = jnp.concatenate([x[:, 0, :], x[:, 1, :]], axis=1) @ p["av_w"] + p["av_b"]

    w_as = (p["gat_w"].reshape(CH, H, CH) * p["att_src"][None]).sum(-1)
    w_ad = (p["gat_w"].reshape(CH, H, CH) * p["att_dst"][None]).sum(-1)

    xh = _mm(feats, p["gat_w"]).reshape(N, H, CH)
    a_src = feats @ w_as
    a_dst = feats @ w_ad

    gmax = jax.nn.leaky_relu(a_src.max(0) + a_dst.max(0), 0.2)

    alpha = jax.nn.leaky_relu(a_src[src] + a_dst[dst], 0.2)
    ex = jnp.exp(alpha - gmax[None, :])
    den = jax.ops.segment_sum(ex, dst, num_segments=N)
    att = ex / den[dst]

    g1 = jax.ops.segment_sum(xh[src] * att[..., None], dst,
                             num_segments=N).reshape(N, H * CH) + p["gat_b"]
    g1 = _bn_relu(g1, p["bn1_g"], p["bn1_b"])

    ones = jnp.ones((E,), jnp.float32)
    deg_in = jax.ops.segment_sum(ones, dst, num_segments=N)
    deg_out = jax.ops.segment_sum(ones, src, num_segments=N)
    deg = jnp.maximum(deg_in, 1.0)[:, None]

    def econv(h, pre):
        d = h.shape[1]
        w1 = p[pre + "_w1"]
        top, bot = w1[:d], w1[d:]
        P = h @ (top - bot) + p[pre + "_b1"]
        Q = h @ bot
        mu = (P * deg_in[:, None]).sum(0) / E + (Q * deg_out[:, None]).sum(0) / E
        cross = (P[dst] * Q[src]).sum(0)
        m2 = ((P * P) * deg_in[:, None]).sum(0) / E \
            + ((Q * Q) * deg_out[:, None]).sum(0) / E + 2.0 * cross / E
        var = m2 - mu * mu
        a = p[pre + "_bng"] / jnp.sqrt(var + 1e-5)
        c = p[pre + "_bnb"] - mu * a
        mn = jnp.maximum((P[dst] + Q[src]) * a + c, 0.0)
        s = jax.ops.segment_sum(mn, dst, num_segments=N)
        return (s @ p[pre + "_w2"]) / deg + p[pre + "_b2"]

    g2 = _bn_relu(econv(g1, "l2"), p["bn2_g"], p["bn2_b"])
    g3 = _bn_relu(econv(g2, "l3") + g2, p["bn3_g"], p["bn3_b"])
    g4 = econv(g3, "l4") + g3

    fc_w = jnp.pad(p["fc_w"], ((0, 0), (0, 126)))
    return _mm(g4, fc_w, block_m=1000)[:, :2] + p["fc_b"]


# speedup vs baseline: 3.6392x; 3.3747x over previous
import functools
import jax
import jax.numpy as jnp
from jax import lax
from jax.experimental import pallas as pl
from jax.experimental.pallas import tpu as pltpu
from jax.experimental.pallas import tpu_sc as plsc

N = 10000
E = 160000
CH = 128
H = 8

_SC_MESH = dict(core_axis_name="c", subcore_axis_name="s")


def _gat_agg(xh2, attF, src, dst):
    kE = 80
    EPT = E // 16
    NB = EPT // kE
    NP = 10240
    RPS = NP // 16
    ZR = 64

    @functools.partial(
        pl.kernel,
        mesh=plsc.VectorSubcoreMesh(**_SC_MESH),
        out_type=jax.ShapeDtypeStruct((NP, H * CH), jnp.float32),
        scratch_types=[
            pltpu.VMEM_SHARED((NP, CH), jnp.float32),
            pltpu.VMEM((kE,), jnp.int32),
            pltpu.VMEM((kE,), jnp.int32),
            pltpu.VMEM((kE,), jnp.int32),
            pltpu.VMEM((kE,), jnp.float32),
            pltpu.VMEM((kE, CH), jnp.float32),
            pltpu.VMEM((ZR, CH), jnp.float32),
            pltpu.SemaphoreType.DMA,
        ],
    )
    def k(xh2_h, attF_h, src_h, dst_h, out_h,
          acc, sbuf, gbuf, dbuf, abuf, rows, zbuf, sem):
        c = lax.axis_index("c")
        s = lax.axis_index("s")
        def zb(i, _):
            for j in range(CH // 16):
                zbuf[i, pl.ds(j * 16, 16)] = jnp.zeros((16,), jnp.float32)
            return 0
        lax.fori_loop(0, ZR, zb, 0)

        def head_pass(hh, _):
            h = c * 4 + hh

            def zc(kk, _):
                pltpu.sync_copy(zbuf, acc.at[pl.ds(s * RPS + kk * ZR, ZR)])
                return 0
            lax.fori_loop(0, RPS // ZR, zc, 0)
            plsc.subcore_barrier()

            hv = lax.broadcast(h * N, (16,))

            def block(b, _):
                ebase = s * EPT + b * kE
                pltpu.sync_copy(src_h.at[pl.ds(ebase, kE)], sbuf)
                pltpu.sync_copy(dst_h.at[pl.ds(ebase, kE)], dbuf)
                pltpu.sync_copy(attF_h.at[pl.ds(h * E + ebase, kE)], abuf)
                for t in range(kE // 16):
                    gbuf[pl.ds(t * 16, 16)] = sbuf[pl.ds(t * 16, 16)] + hv
                pltpu.async_copy(xh2_h.at[gbuf], rows, sem).wait()

                def scale(eg, _):
                    att16 = abuf[pl.ds(eg * 16, 16)]
                    for i in range(16):
                        av = lax.broadcast(att16[i], (16,))
                        e = eg * 16 + i
                        for j in range(CH // 16):
                            sl = pl.ds(j * 16, 16)
                            rows[e, sl] = rows[e, sl] * av
                    return 0
                lax.fori_loop(0, kE // 16, scale, 0)
                pltpu.sync_copy(rows, acc.at[dbuf], add=True)
                return 0
            lax.fori_loop(0, NB, block, 0)
            plsc.subcore_barrier()
            pltpu.sync_copy(
                acc.at[pl.ds(s * RPS, RPS)],
                out_h.at[pl.ds(s * RPS, RPS), pl.ds(h * CH, CH)])
            return 0
        lax.fori_loop(0, 4, head_pass, 0)

    return k(xh2, attF, src, dst)[:N]


def _matmul_kernel(a_ref, b_ref, o_ref):
    o_ref[...] = jnp.dot(a_ref[...], b_ref[...],
                         preferred_element_type=jnp.float32)


def _mm(a, b, block_m=1000):
    m, k = a.shape
    k2, n = b.shape
    grid = (m // block_m,)
    return pl.pallas_call(
        _matmul_kernel,
        grid=grid,
        in_specs=[
            pl.BlockSpec((block_m, k), lambda i: (i, 0)),
            pl.BlockSpec((k, n), lambda i: (0, 0)),
        ],
        out_specs=pl.BlockSpec((block_m, n), lambda i: (i, 0)),
        out_shape=jax.ShapeDtypeStruct((m, n), jnp.float32),
    )(a, b)


def _bn_relu(h, g, b):
    m = h.mean(axis=0)
    v = h.var(axis=0)
    return jnp.maximum((h - m) / jnp.sqrt(v + 1e-5) * g + b, 0.0)


def kernel(x, edge_index, params):
    src = edge_index[0]
    dst = edge_index[1]
    p = params

    feats = jnp.concatenate([x[:, 0, :], x[:, 1, :]], axis=1) @ p["av_w"] + p["av_b"]

    w_as = (p["gat_w"].reshape(CH, H, CH) * p["att_src"][None]).sum(-1)
    w_ad = (p["gat_w"].reshape(CH, H, CH) * p["att_dst"][None]).sum(-1)

    xh = _mm(feats, p["gat_w"]).reshape(N, H, CH)
    a_src = feats @ w_as
    a_dst = feats @ w_ad

    gmax = jax.nn.leaky_relu(a_src.max(0) + a_dst.max(0), 0.2)

    alpha = jax.nn.leaky_relu(a_src[src] + a_dst[dst], 0.2)
    ex = jnp.exp(alpha - gmax[None, :])
    den = jax.ops.segment_sum(ex, dst, num_segments=N)
    att = ex / den[dst]

    xh2 = jnp.swapaxes(xh, 0, 1).reshape(H * N, CH)
    attF = att.T.reshape(-1)
    g1 = _gat_agg(xh2, attF, src, dst) + p["gat_b"]
    g1 = _bn_relu(g1, p["bn1_g"], p["bn1_b"])

    ones = jnp.ones((E,), jnp.float32)
    deg_in = jax.ops.segment_sum(ones, dst, num_segments=N)
    deg_out = jax.ops.segment_sum(ones, src, num_segments=N)
    deg = jnp.maximum(deg_in, 1.0)[:, None]

    def econv(h, pre):
        d = h.shape[1]
        w1 = p[pre + "_w1"]
        top, bot = w1[:d], w1[d:]
        P = h @ (top - bot) + p[pre + "_b1"]
        Q = h @ bot
        mu = (P * deg_in[:, None]).sum(0) / E + (Q * deg_out[:, None]).sum(0) / E
        cross = (P[dst] * Q[src]).sum(0)
        m2 = ((P * P) * deg_in[:, None]).sum(0) / E \
            + ((Q * Q) * deg_out[:, None]).sum(0) / E + 2.0 * cross / E
        var = m2 - mu * mu
        a = p[pre + "_bng"] / jnp.sqrt(var + 1e-5)
        c = p[pre + "_bnb"] - mu * a
        mn = jnp.maximum((P[dst] + Q[src]) * a + c, 0.0)
        s = jax.ops.segment_sum(mn, dst, num_segments=N)
        return (s @ p[pre + "_w2"]) / deg + p[pre + "_b2"]

    g2 = _bn_relu(econv(g1, "l2"), p["bn2_g"], p["bn2_b"])
    g3 = _bn_relu(econv(g2, "l3") + g2, p["bn3_g"], p["bn3_b"])
    g4 = econv(g3, "l4") + g3

    fc_w = jnp.pad(p["fc_w"], ((0, 0), (0, 126)))
    return _mm(g4, fc_w, block_m=1000)[:, :2] + p["fc_b"]


# speedup vs baseline: 5.4334x; 1.4930x over previous
import functools
import jax
import jax.numpy as jnp
from jax import lax
from jax.experimental import pallas as pl
from jax.experimental.pallas import tpu as pltpu
from jax.experimental.pallas import tpu_sc as plsc

N = 10000
E = 160000
CH = 128
H = 8

_SC_MESH = dict(core_axis_name="c", subcore_axis_name="s")


def _gat_agg(xh2, attF, src, dst):
    kE = 80
    EPT = E // 16
    NB = EPT // kE
    NP = 10240
    RPS = NP // 16
    ZR = 64

    @functools.partial(
        pl.kernel,
        mesh=plsc.VectorSubcoreMesh(**_SC_MESH),
        out_type=jax.ShapeDtypeStruct((NP, H * CH), jnp.float32),
        scratch_types=[
            pltpu.VMEM_SHARED((NP, CH), jnp.float32),
            pltpu.VMEM((kE,), jnp.int32),
            pltpu.VMEM((kE,), jnp.int32),
            pltpu.VMEM((kE,), jnp.int32),
            pltpu.VMEM((kE,), jnp.float32),
            pltpu.VMEM((kE, CH), jnp.float32),
            pltpu.VMEM((ZR, CH), jnp.float32),
            pltpu.SemaphoreType.DMA,
        ],
    )
    def k(xh2_h, attF_h, src_h, dst_h, out_h,
          acc, sbuf, gbuf, dbuf, abuf, rows, zbuf, sem):
        c = lax.axis_index("c")
        s = lax.axis_index("s")
        def zb(i, _):
            for j in range(CH // 16):
                zbuf[i, pl.ds(j * 16, 16)] = jnp.zeros((16,), jnp.float32)
            return 0
        lax.fori_loop(0, ZR, zb, 0)

        def head_pass(hh, _):
            h = c * 4 + hh

            def zc(kk, _):
                pltpu.sync_copy(zbuf, acc.at[pl.ds(s * RPS + kk * ZR, ZR)])
                return 0
            lax.fori_loop(0, RPS // ZR, zc, 0)
            plsc.subcore_barrier()

            hv = lax.broadcast(h * N, (16,))

            def block(b, _):
                ebase = s * EPT + b * kE
                pltpu.sync_copy(src_h.at[pl.ds(ebase, kE)], sbuf)
                pltpu.sync_copy(dst_h.at[pl.ds(ebase, kE)], dbuf)
                pltpu.sync_copy(attF_h.at[pl.ds(h * E + ebase, kE)], abuf)
                for t in range(kE // 16):
                    gbuf[pl.ds(t * 16, 16)] = sbuf[pl.ds(t * 16, 16)] + hv
                pltpu.async_copy(xh2_h.at[gbuf], rows, sem).wait()

                def scale(eg, _):
                    att16 = abuf[pl.ds(eg * 16, 16)]
                    for i in range(16):
                        av = lax.broadcast(att16[i], (16,))
                        e = eg * 16 + i
                        for j in range(CH // 16):
                            sl = pl.ds(j * 16, 16)
                            rows[e, sl] = rows[e, sl] * av
                    return 0
                lax.fori_loop(0, kE // 16, scale, 0)
                pltpu.sync_copy(rows, acc.at[dbuf], add=True)
                return 0
            lax.fori_loop(0, NB, block, 0)
            plsc.subcore_barrier()
            pltpu.sync_copy(
                acc.at[pl.ds(s * RPS, RPS)],
                out_h.at[pl.ds(s * RPS, RPS), pl.ds(h * CH, CH)])
            return 0
        lax.fori_loop(0, 4, head_pass, 0)

    return k(xh2, attF, src, dst)[:N]


def _econv_edge_stats(P, Q, src, dst):
    kE = 160
    NBLK = E // kE

    @functools.partial(
        pl.kernel,
        mesh=plsc.VectorSubcoreMesh(**_SC_MESH),
        out_type=jax.ShapeDtypeStruct((32 * 256,), jnp.float32),
        scratch_types=[
            pltpu.VMEM((kE, CH), jnp.float32),
            pltpu.VMEM((kE, CH), jnp.float32),
            pltpu.VMEM((kE,), jnp.int32),
            pltpu.VMEM((kE,), jnp.int32),
            pltpu.VMEM((256,), jnp.float32),
            pltpu.SemaphoreType.DMA,
            pltpu.SemaphoreType.DMA,
        ],
    )
    def k(P_h, Q_h, src_h, dst_h, out_h,
          pbuf, qbuf, sibuf, dibuf, statbuf, sem1, sem2):
        c = lax.axis_index("c")
        s = lax.axis_index("s")
        tid = s * 2 + c
        nb = 31 + (tid < 8).astype(jnp.int32)
        zero = jnp.zeros((16,), jnp.float32)
        init = (zero,) * 16

        def blk(i, carry):
            eb = (tid + i * 32) * kE
            pltpu.sync_copy(src_h.at[pl.ds(eb, kE)], sibuf)
            pltpu.sync_copy(dst_h.at[pl.ds(eb, kE)], dibuf)
            cp1 = pltpu.async_copy(P_h.at[dibuf], pbuf, sem1)
            cp2 = pltpu.async_copy(Q_h.at[sibuf], qbuf, sem2)
            cp1.wait()
            cp2.wait()

            def ed(e, cy):
                sums = list(cy[:8])
                sqs = list(cy[8:])
                for j in range(8):
                    sl = pl.ds(j * 16, 16)
                    m = pbuf[e, sl] + qbuf[e, sl]
                    sums[j] = sums[j] + m
                    sqs[j] = sqs[j] + m * m
                return tuple(sums) + tuple(sqs)
            return lax.fori_loop(0, kE, ed, carry)

        carry = lax.fori_loop(0, nb, blk, init)
        for j in range(8):
            statbuf[pl.ds(j * 16, 16)] = carry[j]
            statbuf[pl.ds(128 + j * 16, 16)] = carry[8 + j]
        pltpu.sync_copy(statbuf, out_h.at[pl.ds(tid * 256, 256)])

    return k(P, Q, src, dst).reshape(32, 2, CH).sum(0)


def _econv_edge_scatter(P, Q, ac, src, dst):
    kE = 80
    NP = 10240
    RPS = NP // 16
    ZR = 64
    HBLK = (E // kE) // 2

    @functools.partial(
        pl.kernel,
        mesh=plsc.VectorSubcoreMesh(**_SC_MESH),
        out_type=jax.ShapeDtypeStruct((2, NP, CH), jnp.float32),
        scratch_types=[
            pltpu.VMEM_SHARED((NP, CH), jnp.float32),
            pltpu.VMEM((kE, CH), jnp.float32),
            pltpu.VMEM((kE, CH), jnp.float32),
            pltpu.VMEM((kE,), jnp.int32),
            pltpu.VMEM((kE,), jnp.int32),
            pltpu.VMEM((256,), jnp.float32),
            pltpu.VMEM((ZR, CH), jnp.float32),
            pltpu.SemaphoreType.DMA,
            pltpu.SemaphoreType.DMA,
        ],
    )
    def k(P_h, Q_h, ac_h, src_h, dst_h, out_h,
          acc, pbuf, qbuf, sibuf, dibuf, acbuf, zbuf, sem1, sem2):
        c = lax.axis_index("c")
        s = lax.axis_index("s")

        def zb(i, _):
            for j in range(8):
                zbuf[i, pl.ds(j * 16, 16)] = jnp.zeros((16,), jnp.float32)
            return 0
        lax.fori_loop(0, ZR, zb, 0)

        def zc(kk, _):
            pltpu.sync_copy(zbuf, acc.at[pl.ds(s * RPS + kk * ZR, ZR)])
            return 0
        lax.fori_loop(0, RPS // ZR, zc, 0)
        pltpu.sync_copy(ac_h, acbuf)
        avs = [acbuf[pl.ds(j * 16, 16)] for j in range(8)]
        cvs = [acbuf[pl.ds(128 + j * 16, 16)] for j in range(8)]
        plsc.subcore_barrier()

        nb = 62 + (s < 8).astype(jnp.int32)

        def blk(i, _):
            eb = (c * HBLK + s + i * 16) * kE
            pltpu.sync_copy(src_h.at[pl.ds(eb, kE)], sibuf)
            pltpu.sync_copy(dst_h.at[pl.ds(eb, kE)], dibuf)
            cp1 = pltpu.async_copy(P_h.at[dibuf], pbuf, sem1)
            cp2 = pltpu.async_copy(Q_h.at[sibuf], qbuf, sem2)
            cp1.wait()
            cp2.wait()

            def ed(e, _2):
                for j in range(8):
                    sl = pl.ds(j * 16, 16)
                    m = pbuf[e, sl] + qbuf[e, sl]
                    pbuf[e, sl] = jnp.maximum(m * avs[j] + cvs[j], 0.0)
                return 0
            lax.fori_loop(0, kE, ed, 0)
            pltpu.sync_copy(pbuf, acc.at[dibuf], add=True)
            return 0
        lax.fori_loop(0, nb, blk, 0)
        plsc.subcore_barrier()
        pltpu.sync_copy(acc.at[pl.ds(s * RPS, RPS)],
                        out_h.at[c, pl.ds(s * RPS, RPS)])

    out = k(P, Q, ac, src, dst)
    return (out[0] + out[1])[:N]


def _matmul_kernel(a_ref, b_ref, o_ref):
    o_ref[...] = jnp.dot(a_ref[...], b_ref[...],
                         preferred_element_type=jnp.float32)


def _mm(a, b, block_m=1000):
    m, k = a.shape
    k2, n = b.shape
    grid = (m // block_m,)
    return pl.pallas_call(
        _matmul_kernel,
        grid=grid,
        in_specs=[
            pl.BlockSpec((block_m, k), lambda i: (i, 0)),
            pl.BlockSpec((k, n), lambda i: (0, 0)),
        ],
        out_specs=pl.BlockSpec((block_m, n), lambda i: (i, 0)),
        out_shape=jax.ShapeDtypeStruct((m, n), jnp.float32),
    )(a, b)


def _bn_relu(h, g, b):
    m = h.mean(axis=0)
    v = h.var(axis=0)
    return jnp.maximum((h - m) / jnp.sqrt(v + 1e-5) * g + b, 0.0)


def kernel(x, edge_index, params):
    src = edge_index[0]
    dst = edge_index[1]
    p = params

    feats = jnp.concatenate([x[:, 0, :], x[:, 1, :]], axis=1) @ p["av_w"] + p["av_b"]

    w_as = (p["gat_w"].reshape(CH, H, CH) * p["att_src"][None]).sum(-1)
    w_ad = (p["gat_w"].reshape(CH, H, CH) * p["att_dst"][None]).sum(-1)

    xh = _mm(feats, p["gat_w"]).reshape(N, H, CH)
    a_src = feats @ w_as
    a_dst = feats @ w_ad

    gmax = jax.nn.leaky_relu(a_src.max(0) + a_dst.max(0), 0.2)

    alpha = jax.nn.leaky_relu(a_src[src] + a_dst[dst], 0.2)
    ex = jnp.exp(alpha - gmax[None, :])
    den = jax.ops.segment_sum(ex, dst, num_segments=N)
    att = ex / den[dst]

    xh2 = jnp.swapaxes(xh, 0, 1).reshape(H * N, CH)
    attF = att.T.reshape(-1)
    g1 = _gat_agg(xh2, attF, src, dst) + p["gat_b"]
    g1 = _bn_relu(g1, p["bn1_g"], p["bn1_b"])

    ones = jnp.ones((E,), jnp.float32)
    deg_in = jax.ops.segment_sum(ones, dst, num_segments=N)
    deg = jnp.maximum(deg_in, 1.0)[:, None]

    def econv(h, pre):
        d = h.shape[1]
        w1 = p[pre + "_w1"]
        top, bot = w1[:d], w1[d:]
        P = h @ (top - bot) + p[pre + "_b1"]
        Q = h @ bot
        st = _econv_edge_stats(P, Q, src, dst)
        mu = st[0] / E
        var = st[1] / E - mu * mu
        a = p[pre + "_bng"] / jnp.sqrt(var + 1e-5)
        c = p[pre + "_bnb"] - mu * a
        s = _econv_edge_scatter(P, Q, jnp.concatenate([a, c]), src, dst)
        return (s @ p[pre + "_w2"]) / deg + p[pre + "_b2"]

    g2 = _bn_relu(econv(g1, "l2"), p["bn2_g"], p["bn2_b"])
    g3 = _bn_relu(econv(g2, "l3") + g2, p["bn3_g"], p["bn3_b"])
    g4 = econv(g3, "l4") + g3

    fc_w = jnp.pad(p["fc_w"], ((0, 0), (0, 126)))
    return _mm(g4, fc_w, block_m=1000)[:, :2] + p["fc_b"]
